# Initial kernel scaffold; baseline (speedup 1.0000x reference)
import jax
import jax.numpy as jnp
from jax.experimental import pallas as pl


def kernel(f_atoms, f_bonds, f_mol, W_i, W_h, W_o, a2b, b2a, b2revb, ascope):
    raise NotImplementedError("write your pallas kernel here")


# speedup vs baseline: 2.1961x; 2.1961x over previous
import functools

import jax
import jax.numpy as jnp
from jax import lax
from jax.experimental import pallas as pl
from jax.experimental.pallas import tpu as pltpu
from jax.experimental.pallas import tpu_sc as plsc

F32 = jnp.float32

N_ATOMS = 50000
N_BONDS = 800000
MAX_NB = 16
FA = 133
FB = 147
H = 100
HP = 128
N_MOLS = 2500
APM = 20
DEPTH = 3

NC = 2
NS = 16
NW = NC * NS

BA = 400
BB = 320

BM = 1600
BMA = 2000

_SC_MESH = plsc.VectorSubcoreMesh(core_axis_name="c", subcore_axis_name="s")


def _wid():
    return lax.axis_index("s") * NC + lax.axis_index("c")


@functools.partial(
    pl.kernel,
    out_type=jax.ShapeDtypeStruct((N_ATOMS, HP), F32),
    mesh=_SC_MESH,
    scratch_types=[
        pltpu.VMEM((MAX_NB * BA,), jnp.int32),
        pltpu.VMEM((BA, HP), F32),
        pltpu.SemaphoreType.DMA,
    ],
)
def _sc_segsum(m_hbm, a2bf_hbm, out_hbm, idx_v, acc_v, sem):
    wid = _wid()
    nblk = N_ATOMS // BA
    nmine = (nblk - wid + NW - 1) // NW

    def body(i, carry):
        base = (wid + NW * i) * BA
        for k in range(MAX_NB):
            pltpu.sync_copy(a2bf_hbm.at[pl.ds(k * N_ATOMS + base, BA)],
                            idx_v.at[pl.ds(k * BA, BA)])
        pltpu.async_copy(m_hbm.at[idx_v.at[pl.ds(0, BA)]], acc_v, sem).wait()
        cps = [
            pltpu.async_copy(m_hbm.at[idx_v.at[pl.ds(k * BA, BA)]], acc_v, sem,
                             add=True)
            for k in range(1, MAX_NB)
        ]
        for c in cps:
            c.wait()
        pltpu.sync_copy(acc_v, out_hbm.at[pl.ds(base, BA)])
        return carry

    lax.fori_loop(0, nmine, body, 0)


@functools.partial(
    pl.kernel,
    out_type=(
        jax.ShapeDtypeStruct((N_BONDS, HP), F32),
        jax.ShapeDtypeStruct((N_BONDS, HP), F32),
    ),
    mesh=_SC_MESH,
    scratch_types=[
        pltpu.VMEM((BB,), jnp.int32),
        pltpu.VMEM((BB,), jnp.int32),
        pltpu.VMEM((BB, HP), F32),
        pltpu.VMEM((BB, HP), F32),
        pltpu.SemaphoreType.DMA,
    ],
)
def _sc_gather2(a_hbm, m_hbm, b2a_hbm, b2revb_hbm, ag_hbm, mg_hbm,
                ia_v, ib_v, bufa_v, bufm_v, sem):
    wid = _wid()
    nblk = N_BONDS // BB
    nmine = (nblk - wid + NW - 1) // NW

    def body(i, carry):
        base = (wid + NW * i) * BB
        pltpu.sync_copy(b2a_hbm.at[pl.ds(base, BB)], ia_v)
        pltpu.sync_copy(b2revb_hbm.at[pl.ds(base, BB)], ib_v)
        ca = pltpu.async_copy(a_hbm.at[ia_v], bufa_v, sem)
        cb = pltpu.async_copy(m_hbm.at[ib_v], bufm_v, sem)
        ca.wait()
        cb.wait()
        pltpu.sync_copy(bufa_v, ag_hbm.at[pl.ds(base, BB)])
        pltpu.sync_copy(bufm_v, mg_hbm.at[pl.ds(base, BB)])
        return carry

    lax.fori_loop(0, nmine, body, 0)


def _t1_body(x_ref, w_ref, inp_ref, m0_ref):
    y = lax.dot_general(x_ref[...], w_ref[...], (((1,), (1,)), ((), ())),
                        preferred_element_type=F32)
    inp_ref[...] = y
    m0_ref[...] = jnp.maximum(y, 0.0)


def _tc_input(f_bonds, wi_p):
    return pl.pallas_call(
        _t1_body,
        grid=(N_BONDS // BM,),
        in_specs=[
            pl.BlockSpec((BM, FB), lambda i: (i, 0)),
            pl.BlockSpec((HP, FB), lambda i: (0, 0)),
        ],
        out_specs=[
            pl.BlockSpec((BM, HP), lambda i: (i, 0)),
            pl.BlockSpec((BM, HP), lambda i: (i, 0)),
        ],
        out_shape=[
            jax.ShapeDtypeStruct((N_BONDS, HP), F32),
            jax.ShapeDtypeStruct((N_BONDS, HP), F32),
        ],
    )(f_bonds, wi_p)


def _t2_body(inp_ref, ag_ref, mg_ref, w_ref, out_ref):
    x = ag_ref[...] - mg_ref[...]
    y = lax.dot_general(x, w_ref[...], (((1,), (1,)), ((), ())),
                        preferred_element_type=F32)
    out_ref[...] = jnp.maximum(inp_ref[...] + y, 0.0)


def _tc_update(inp, ag, mg, wh_p):
    return pl.pallas_call(
        _t2_body,
        grid=(N_BONDS // BM,),
        in_specs=[
            pl.BlockSpec((BM, HP), lambda i: (i, 0)),
            pl.BlockSpec((BM, HP), lambda i: (i, 0)),
            pl.BlockSpec((BM, HP), lambda i: (i, 0)),
            pl.BlockSpec((HP, HP), lambda i: (0, 0)),
        ],
        out_specs=pl.BlockSpec((BM, HP), lambda i: (i, 0)),
        out_shape=jax.ShapeDtypeStruct((N_BONDS, HP), F32),
    )(inp, ag, mg, wh_p)


def _t4_body(fa_ref, a_ref, woa_ref, wom_ref, st_ref, ah_ref, mol_ref):
    y = lax.dot_general(fa_ref[...], woa_ref[...], (((1,), (1,)), ((), ())),
                        preferred_element_type=F32)
    y = y + lax.dot_general(a_ref[:, :H], wom_ref[...], (((1,), (1,)), ((), ())),
                            preferred_element_type=F32)
    ah = jnp.maximum(y, 0.0)
    ah_ref[...] = ah
    mol_ref[...] = lax.dot_general(st_ref[...], ah, (((1,), (0,)), ((), ())),
                                   preferred_element_type=F32)[None]


def _tc_readout(f_atoms, a_msg, wo_a, wo_m, st):
    return pl.pallas_call(
        _t4_body,
        grid=(N_ATOMS // BMA,),
        in_specs=[
            pl.BlockSpec((BMA, FA), lambda i: (i, 0)),
            pl.BlockSpec((BMA, HP), lambda i: (i, 0)),
            pl.BlockSpec((H, FA), lambda i: (0, 0)),
            pl.BlockSpec((H, H), lambda i: (0, 0)),
            pl.BlockSpec((BMA // APM, BMA), lambda i: (0, 0)),
        ],
        out_specs=[
            pl.BlockSpec((BMA, H), lambda i: (i, 0)),
            pl.BlockSpec((1, BMA // APM, H), lambda i: (i, 0, 0)),
        ],
        out_shape=[
            jax.ShapeDtypeStruct((N_ATOMS, H), F32),
            jax.ShapeDtypeStruct((N_ATOMS // BMA, BMA // APM, H), F32),
        ],
    )(f_atoms, a_msg, wo_a, wo_m, st)


def kernel(f_atoms, f_bonds, f_mol, W_i, W_h, W_o, a2b, b2a, b2revb, ascope):
    wi_p = jnp.zeros((HP, FB), F32).at[:H].set(W_i)
    wh_p = jnp.zeros((HP, HP), F32).at[:H, :H].set(W_h)
    wo_a = W_o[:, :FA]
    wo_m = W_o[:, FA:]
    a2bf = a2b.T.reshape(-1).astype(jnp.int32)
    b2a32 = b2a.astype(jnp.int32)
    b2revb32 = b2revb.astype(jnp.int32)

    inp, msg = _tc_input(f_bonds, wi_p)
    for _ in range(DEPTH - 1):
        a_msg = _sc_segsum(msg, a2bf)
        ag, mg = _sc_gather2(a_msg, msg, b2a32, b2revb32)
        msg = _tc_update(inp, ag, mg, wh_p)
    a_msg = _sc_segsum(msg, a2bf)

    st = jnp.repeat(jnp.eye(BMA // APM, dtype=F32), APM, axis=1)
    atom_hiddens, mol_sum = _tc_readout(f_atoms, a_msg, wo_a, wo_m, st)
    mol_sum = mol_sum.reshape(N_MOLS, H)
    sizes = ascope[:, 1].astype(F32)
    mol_vecs = jnp.concatenate([mol_sum / sizes[:, None], f_mol], axis=1)
    return (mol_vecs, atom_hiddens)
